# baseline (speedup 1.0000x reference)
import functools

import jax
import jax.numpy as jnp
from jax import lax
from jax.experimental import pallas as pl
from jax.experimental.pallas import tpu as pltpu
from jax.experimental.pallas import tpu_sc as plsc

F32 = jnp.float32
I32 = jnp.int32

_SC_CORES = 2
_SC_TILES = 16
_LANES = 16


def _dot(a, b):
    return jnp.dot(a, b, preferred_element_type=F32)


def _lin_body(x_ref, w_ref, b_ref, o_ref):
    o_ref[...] = _dot(x_ref[...], w_ref[...]) + b_ref[...]


def _resid_ff_body(x_ref, wl_ref, bl_ref, w0_ref, b0_ref, w1_ref, b1_ref, o_ref):
    x = x_ref[...]
    h = jnp.maximum(_dot(x, w0_ref[...]) + b0_ref[...], 0.0)
    o_ref[...] = _dot(x, wl_ref[...]) + bl_ref[...] + _dot(h, w1_ref[...]) + b1_ref[...]


def _node_stage_body(x_ref, wl_ref, bl_ref, w0_ref, b0_ref, w1_ref, b1_ref,
                     res_ref, meta_ref):
    x = x_ref[...]
    h = jnp.maximum(_dot(x, w0_ref[...]) + b0_ref[...], 0.0)
    mlp = _dot(h, w1_ref[...]) + b1_ref[...]
    res_ref[...] = mlp
    meta_ref[...] = _dot(x, wl_ref[...]) + bl_ref[...] + mlp


def _conv_tail_body(x_ref, agg_ref, add_ref, wu1_ref, wu2_ref, bu_ref,
                    wr_ref, br_ref, g_ref, b_ref, o_ref, *, has_add):
    out = jnp.maximum(
        _dot(x_ref[...], wu1_ref[...]) + _dot(agg_ref[...], wu2_ref[...])
        + bu_ref[...], 0.0)
    if has_add:
        out = out + add_ref[...]
    out = out + jnp.maximum(_dot(out, wr_ref[...]) + br_ref[...], 0.0)
    mu = jnp.mean(out, axis=-1, keepdims=True)
    d = out - mu
    var = jnp.mean(d * d, axis=-1, keepdims=True)
    o_ref[...] = d * jax.lax.rsqrt(var + 1e-5) * g_ref[...] + b_ref[...]


def _pick_bn(n):
    for bn in (2048, 2000, 1024, 1000, 512, 500, 256, 200, 128, 100, 8):
        if n % bn == 0:
            return bn
    return n


def _row_call(body, x_args, w_args, n_out, out_dims):
    n = x_args[0].shape[0]
    bn = _pick_bn(n)
    grid = (n // bn,)
    in_specs = (
        [pl.BlockSpec((bn, a.shape[1]), lambda i: (i, 0)) for a in x_args]
        + [pl.BlockSpec(a.shape, lambda i: tuple(0 for _ in a.shape))
           for a in w_args]
    )
    out_specs = [pl.BlockSpec((bn, d), lambda i: (i, 0)) for d in out_dims]
    out_shape = [jax.ShapeDtypeStruct((n, d), F32) for d in out_dims]
    if n_out == 1:
        out_specs, out_shape = out_specs[0], out_shape[0]
    return pl.pallas_call(
        body, grid=grid, in_specs=in_specs, out_specs=out_specs,
        out_shape=out_shape)(*x_args, *w_args)


def _r2(b):
    return b.reshape(1, -1)


def _linear(x, w, b):
    return _row_call(_lin_body, [x], [w, _r2(b)], 1, [w.shape[1]])


def _residual_ff(x, lin_p, ff_p):
    (wl, bl), ((w0, b0), (w1, b1)) = lin_p, ff_p
    return _row_call(_resid_ff_body, [x],
                     [wl, _r2(bl), w0, _r2(b0), w1, _r2(b1)], 1, [wl.shape[1]])


def _node_stage(x, lin_p, ff_p):
    (wl, bl), ((w0, b0), (w1, b1)) = lin_p, ff_p
    return _row_call(_node_stage_body, [x],
                     [wl, _r2(bl), w0, _r2(b0), w1, _r2(b1)], 2,
                     [wl.shape[1], wl.shape[1]])


def _conv_tail(x, agg, add, p):
    wu, bu = p['upd']
    h = wu.shape[1]
    wu1, wu2 = wu[:h], wu[h:]
    wr, br = p['resid'][0]
    has_add = add is not None
    body = functools.partial(_conv_tail_body, has_add=has_add)
    if not has_add:
        add = jnp.zeros((x.shape[0], 1), F32)
    return _row_call(body, [x, agg, add],
                     [wu1, wu2, _r2(bu), wr, _r2(br), _r2(p['ln_g']),
                      _r2(p['ln_b'])], 1, [h])


_SCAN_B = 2000
_BLK = 400
_ZROWS = 40

_SC_PARAMS = pltpu.CompilerParams(needs_layout_passes=False,
                                  use_tc_tiling_on_sc=False)


def _make_compact(n_nodes, n_edges, n_chunks):
    ept = n_edges // _SC_TILES
    cap = ept
    nc = n_nodes // n_chunks
    kpc = n_chunks // _SC_CORES
    mesh = plsc.VectorSubcoreMesh(core_axis_name="c", subcore_axis_name="s")
    out_type = tuple(
        jax.ShapeDtypeStruct((n_chunks * _SC_TILES * cap,), I32)
        for _ in range(3)
    ) + (jax.ShapeDtypeStruct((n_chunks * _SC_TILES * _LANES,), I32),)
    pad = cap + 2 * _LANES
    scratch = [
        pltpu.VMEM((_SCAN_B,), I32),
        pltpu.VMEM((_SCAN_B,), I32),
        pltpu.VMEM((pad,), I32),
        pltpu.VMEM((pad,), I32),
        pltpu.VMEM((pad,), I32),
        pltpu.VMEM((_LANES,), I32),
    ]
    trash = cap + _LANES

    def body(src_h, dst_h, osrc, odst, oeid, ocnt,
             sblk, dblk, ssel, dsel, esel, cnt_v):
        c = lax.axis_index("c")
        t = lax.axis_index("s")
        e0 = t * ept
        lane = lax.iota(I32, _LANES)
        for kk in range(kpc):
            chunk = c * kpc + kk
            lo = chunk * nc
            region = (chunk * _SC_TILES + t) * cap

            def vstep(j, cur, base=None):
                dv = dblk[pl.ds(j * _LANES, _LANES)]
                sv = sblk[pl.ds(j * _LANES, _LANES)]
                dl = dv - lo
                mask = (dl >= 0) & (dl < nc)
                ev = base + j * _LANES + lane
                pc = plsc.cumsum(mask.astype(I32))
                pos = jnp.where(mask, cur + pc - 1, trash)
                plsc.store_scatter(dsel, [pos], dl)
                plsc.store_scatter(ssel, [pos], sv)
                plsc.store_scatter(esel, [pos], ev)
                return cur + jnp.max(pc, axis=0)

            def scan_blk(bi, cur):
                base = e0 + bi * _SCAN_B
                pltpu.sync_copy(src_h.at[pl.ds(base, _SCAN_B)], sblk)
                pltpu.sync_copy(dst_h.at[pl.ds(base, _SCAN_B)], dblk)
                return lax.fori_loop(
                    0, _SCAN_B // _LANES,
                    functools.partial(vstep, base=base), cur)

            cur = lax.fori_loop(0, ept // _SCAN_B, scan_blk, 0)
            padn = ((cur + _BLK - 1) // _BLK) * _BLK
            dummy = jnp.full((_LANES,), nc, I32)
            zero16 = jnp.zeros((_LANES,), I32)

            def padstep(cur):
                dsel[pl.ds(cur, _LANES)] = dummy
                ssel[pl.ds(cur, _LANES)] = zero16
                esel[pl.ds(cur, _LANES)] = zero16
                return cur + _LANES

            lax.while_loop(lambda u: u < padn, padstep, cur)
            roff = pl.multiple_of(region, 8)
            pltpu.sync_copy(ssel.at[pl.ds(0, cap)], osrc.at[pl.ds(roff, cap)])
            pltpu.sync_copy(dsel.at[pl.ds(0, cap)], odst.at[pl.ds(roff, cap)])
            pltpu.sync_copy(esel.at[pl.ds(0, cap)], oeid.at[pl.ds(roff, cap)])
            cnt_v[...] = jnp.full((_LANES,), padn // _BLK, I32)
            coff = pl.multiple_of((chunk * _SC_TILES + t) * _LANES, 8)
            pltpu.sync_copy(cnt_v, ocnt.at[pl.ds(coff, _LANES)])

    return pl.kernel(body, out_type=out_type, mesh=mesh,
                     scratch_types=scratch, compiler_params=_SC_PARAMS)


def _make_agg(n_nodes, n_edges, n_chunks, hid):
    ept = n_edges // _SC_TILES
    cap = ept
    nc = n_nodes // n_chunks
    kpc = n_chunks // _SC_CORES
    nq = hid // _LANES
    nzb = nc // _ZROWS
    mesh = plsc.VectorSubcoreMesh(core_axis_name="c", subcore_axis_name="s")
    out_type = jax.ShapeDtypeStruct((n_nodes, hid), F32)
    scratch = [
        pltpu.VMEM_SHARED((nc + _LANES, hid), F32),
        pltpu.VMEM((_BLK, hid), F32),
        pltpu.VMEM((_BLK, hid), F32),
        [pltpu.VMEM((_BLK,), I32) for _ in range(3)],
        [pltpu.VMEM((_BLK,), I32) for _ in range(3)],
        pltpu.VMEM((_LANES,), I32),
        pltpu.SemaphoreType.DMA,
        pltpu.SemaphoreType.DMA,
        pltpu.SemaphoreType.DMA,
        pltpu.SemaphoreType.DMA,
        pltpu.SemaphoreType.DMA,
        pltpu.SemaphoreType.DMA,
    ]

    def body(pre_s, pre_e, lsrc, ldst, leid, lcnt, out,
             agg, buf, buf2, set0, set1, cnt_v, lsem0, lsem1, esem, ssem,
             esem2, ssem2):
        c = lax.axis_index("c")
        t = lax.axis_index("s")
        zv = jnp.zeros((_LANES,), F32)
        sets = (set0, set1)
        lsems = (lsem0, lsem1)
        hbm_lists = (lsrc, ldst, leid)

        def issue_lists(ph, base):
            for hl, vb in zip(hbm_lists, sets[ph]):
                pltpu.async_copy(hl.at[pl.ds(base, _BLK)], vb, lsems[ph])

        def wait_lists(ph):
            for hl, vb in zip(hbm_lists, sets[ph]):
                pltpu.make_async_copy(hl.at[pl.ds(0, _BLK)], vb,
                                      lsems[ph]).wait()

        for kk in range(kpc):
            chunk = c * kpc + kk
            region = (chunk * _SC_TILES + t) * cap
            for r in range(_ZROWS):
                for q in range(nq):
                    buf[r, pl.ds(q * _LANES, _LANES)] = zv
            for b in range(-(-nzb // _SC_TILES)):
                bid = t + b * _SC_TILES
                @pl.when(bid < nzb)
                def _():
                    pltpu.sync_copy(buf.at[pl.ds(0, _ZROWS)],
                                    agg.at[pl.ds(bid * _ZROWS, _ZROWS)])
            plsc.subcore_barrier()

            coff = pl.multiple_of((chunk * _SC_TILES + t) * _LANES, 8)
            pltpu.sync_copy(lcnt.at[pl.ds(coff, _LANES)], cnt_v)
            nblk = jnp.max(cnt_v[...], axis=0)

            @pl.when(nblk > 0)
            def _():
                issue_lists(0, pl.multiple_of(region, 8))

            def pair(pr, carry):
                for ph in range(2):
                    b = 2 * pr + ph
                    sidx, didx, eidx = sets[ph]

                    @pl.when(b < nblk)
                    def _():
                        wait_lists(ph)

                        @pl.when(b + 1 < nblk)
                        def _():
                            issue_lists(
                                1 - ph,
                                pl.multiple_of(region + (b + 1) * _BLK, 8))

                        hb = _BLK // 2
                        d0 = pltpu.async_copy(
                            pre_e.at[eidx.at[pl.ds(0, hb)]],
                            buf.at[pl.ds(0, hb)], esem)
                        d1 = pltpu.async_copy(
                            pre_e.at[eidx.at[pl.ds(hb, hb)]],
                            buf.at[pl.ds(hb, hb)], esem2)
                        d2 = pltpu.async_copy(
                            pre_s.at[sidx.at[pl.ds(0, hb)]],
                            buf2.at[pl.ds(0, hb)], ssem)
                        d3 = pltpu.async_copy(
                            pre_s.at[sidx.at[pl.ds(hb, hb)]],
                            buf2.at[pl.ds(hb, hb)], ssem2)
                        d0.wait()
                        d1.wait()
                        d2.wait()
                        d3.wait()

                        @plsc.parallel_loop(0, _BLK, 1, unroll=8)
                        def _(i):
                            for q in range(nq):
                                sl = pl.ds(q * _LANES, _LANES)
                                buf[i, sl] = jnp.maximum(
                                    buf[i, sl] + buf2[i, sl], 0.0)

                        pltpu.sync_copy(buf, agg.at[didx], add=True)
                return carry

            lax.fori_loop(0, (nblk + 1) // 2, pair, 0)
            plsc.subcore_barrier()

            for b in range(-(-nzb // _SC_TILES)):
                bid = t + b * _SC_TILES
                @pl.when(bid < nzb)
                def _():
                    r0 = bid * _ZROWS
                    pltpu.sync_copy(
                        agg.at[pl.ds(r0, _ZROWS)],
                        out.at[pl.ds(pl.multiple_of(chunk * nc + r0, 8),
                                     _ZROWS)])
            plsc.subcore_barrier()

    return pl.kernel(body, out_type=out_type, mesh=mesh,
                     scratch_types=scratch, compiler_params=_SC_PARAMS)


def _conv(p, x, graph, pre_e, add):
    wm, _ = p['msg']
    h = wm.shape[1]
    pre_s = _linear(x, wm[:h], jnp.zeros((h,), F32))
    agg = graph['agg_fn'](pre_s, pre_e, *graph['lists'])
    return _conv_tail(x, agg, add, p)


def _pre_edge(ea, p):
    wm, bm = p['msg']
    h = wm.shape[1]
    return _linear(ea, wm[h:], bm)


def kernel(node_feature, edge_index, edge_feature, line_node_feature,
           line_edge_index, line_edge_feature, global_state, group_size,
           cell, coords, params):
    p = params
    n_nodes = node_feature.shape[0]
    batch = global_state.shape[0]

    src, dst = edge_index[0], edge_index[1]
    lsrc, ldst = line_edge_index[0], line_edge_index[1]

    node_res, meta_node = _node_stage(node_feature, p['node_linear'],
                                      p['node_ff'])
    meta_edge = _residual_ff(edge_feature, p['edge_linear'], p['edge_ff'])
    line_meta_node = _residual_ff(line_node_feature, p['line_node_linear'],
                                  p['line_node_ff'])
    line_meta_edge = _residual_ff(line_edge_feature, p['line_edge_linear'],
                                  p['line_edge_ff'])

    def tiny_residual(f, lin_p, ff_p):
        (wl, bl), ff = lin_p, ff_p
        y = f @ wl + bl
        hdn = f
        for (w0, b0) in ff[:-1]:
            hdn = jnp.maximum(hdn @ w0 + b0, 0.0)
        return y + hdn @ ff[-1][0] + ff[-1][1]

    meta_state = tiny_residual(global_state, p['state_linear'], p['state_ff'])
    meta_state = jnp.repeat(meta_state, group_size,
                            total_repeat_length=n_nodes)
    meta_cell = tiny_residual(cell, p['cell_linear'], p['cell_ff'])
    cell_mean = jnp.mean(meta_cell)
    meta_coords = _residual_ff(
        jnp.pad(coords, ((0, 0), (0, 5))),
        (jnp.pad(p['coord_linear'][0], ((0, 5), (0, 0))), p['coord_linear'][1]),
        ((jnp.pad(p['coord_ff'][0][0], ((0, 5), (0, 0))), p['coord_ff'][0][1]),
         p['coord_ff'][1]))

    crystal_add = meta_state[:, None] + meta_coords + cell_mean

    hid = 64
    n_ln = line_node_feature.shape[0]
    line_graph = {
        'lists': _make_compact(n_ln, lsrc.shape[0], 10)(lsrc, ldst),
        'agg_fn': _make_agg(n_ln, lsrc.shape[0], 10, hid),
    }
    node_graph = {
        'lists': _make_compact(n_nodes, src.shape[0], 2)(src, dst),
        'agg_fn': _make_agg(n_nodes, src.shape[0], 2, hid),
    }

    n_layers = len(p['gcs'])
    for i in range(n_layers):
        line_pre_e = _pre_edge(line_meta_edge, p['line_gcs'][i])
        line_meta_node = _conv(p['line_gcs'][i], line_meta_node, line_graph,
                               line_pre_e, None)
        pre_e = _pre_edge(meta_edge, p['gcs'][i])
        meta_node = _conv(p['gcs'][i], meta_node, node_graph, pre_e,
                          crystal_add)

    fp = p['final_gcs'][n_layers - 1]
    final_pre_e = _pre_edge(line_meta_node, fp)
    final_node = _conv(fp, meta_node, node_graph, final_pre_e, crystal_add)
    return node_res + final_node


# speedup vs baseline: 1.0030x; 1.0030x over previous
import functools

import jax
import jax.numpy as jnp
from jax import lax
from jax.experimental import pallas as pl
from jax.experimental.pallas import tpu as pltpu
from jax.experimental.pallas import tpu_sc as plsc

F32 = jnp.float32
I32 = jnp.int32

_SC_CORES = 2
_SC_TILES = 16
_LANES = 16


def _dot(a, b):
    return jnp.dot(a, b, preferred_element_type=F32)


def _lin_body(x_ref, w_ref, b_ref, o_ref):
    o_ref[...] = _dot(x_ref[...], w_ref[...]) + b_ref[...]


def _resid_ff_body(x_ref, wl_ref, bl_ref, w0_ref, b0_ref, w1_ref, b1_ref, o_ref):
    x = x_ref[...]
    h = jnp.maximum(_dot(x, w0_ref[...]) + b0_ref[...], 0.0)
    o_ref[...] = _dot(x, wl_ref[...]) + bl_ref[...] + _dot(h, w1_ref[...]) + b1_ref[...]


def _node_stage_body(x_ref, wl_ref, bl_ref, w0_ref, b0_ref, w1_ref, b1_ref,
                     res_ref, meta_ref):
    x = x_ref[...]
    h = jnp.maximum(_dot(x, w0_ref[...]) + b0_ref[...], 0.0)
    mlp = _dot(h, w1_ref[...]) + b1_ref[...]
    res_ref[...] = mlp
    meta_ref[...] = _dot(x, wl_ref[...]) + bl_ref[...] + mlp


def _conv_tail_body(x_ref, agg_ref, add_ref, wu1_ref, wu2_ref, bu_ref,
                    wr_ref, br_ref, g_ref, b_ref, o_ref, *, has_add):
    out = jnp.maximum(
        _dot(x_ref[...], wu1_ref[...]) + _dot(agg_ref[...], wu2_ref[...])
        + bu_ref[...], 0.0)
    if has_add:
        out = out + add_ref[...]
    out = out + jnp.maximum(_dot(out, wr_ref[...]) + br_ref[...], 0.0)
    mu = jnp.mean(out, axis=-1, keepdims=True)
    d = out - mu
    var = jnp.mean(d * d, axis=-1, keepdims=True)
    o_ref[...] = d * jax.lax.rsqrt(var + 1e-5) * g_ref[...] + b_ref[...]


def _pick_bn(n):
    for bn in (2048, 2000, 1024, 1000, 512, 500, 256, 200, 128, 100, 8):
        if n % bn == 0:
            return bn
    return n


def _row_call(body, x_args, w_args, n_out, out_dims):
    n = x_args[0].shape[0]
    bn = _pick_bn(n)
    grid = (n // bn,)
    in_specs = (
        [pl.BlockSpec((bn, a.shape[1]), lambda i: (i, 0)) for a in x_args]
        + [pl.BlockSpec(a.shape, lambda i: tuple(0 for _ in a.shape))
           for a in w_args]
    )
    out_specs = [pl.BlockSpec((bn, d), lambda i: (i, 0)) for d in out_dims]
    out_shape = [jax.ShapeDtypeStruct((n, d), F32) for d in out_dims]
    if n_out == 1:
        out_specs, out_shape = out_specs[0], out_shape[0]
    return pl.pallas_call(
        body, grid=grid, in_specs=in_specs, out_specs=out_specs,
        out_shape=out_shape)(*x_args, *w_args)


def _r2(b):
    return b.reshape(1, -1)


def _linear(x, w, b):
    return _row_call(_lin_body, [x], [w, _r2(b)], 1, [w.shape[1]])


def _residual_ff(x, lin_p, ff_p):
    (wl, bl), ((w0, b0), (w1, b1)) = lin_p, ff_p
    return _row_call(_resid_ff_body, [x],
                     [wl, _r2(bl), w0, _r2(b0), w1, _r2(b1)], 1, [wl.shape[1]])


def _node_stage(x, lin_p, ff_p):
    (wl, bl), ((w0, b0), (w1, b1)) = lin_p, ff_p
    return _row_call(_node_stage_body, [x],
                     [wl, _r2(bl), w0, _r2(b0), w1, _r2(b1)], 2,
                     [wl.shape[1], wl.shape[1]])


def _conv_tail(x, agg, add, p):
    wu, bu = p['upd']
    h = wu.shape[1]
    wu1, wu2 = wu[:h], wu[h:]
    wr, br = p['resid'][0]
    has_add = add is not None
    body = functools.partial(_conv_tail_body, has_add=has_add)
    if not has_add:
        add = jnp.zeros((x.shape[0], 1), F32)
    return _row_call(body, [x, agg, add],
                     [wu1, wu2, _r2(bu), wr, _r2(br), _r2(p['ln_g']),
                      _r2(p['ln_b'])], 1, [h])


_SCAN_B = 2000
_BLK = 400
_ZROWS = 40

_SC_PARAMS = pltpu.CompilerParams(needs_layout_passes=False,
                                  use_tc_tiling_on_sc=False)


def _make_compact(n_nodes, n_edges, n_chunks):
    ept = n_edges // _SC_TILES
    cap = ept
    nc = n_nodes // n_chunks
    kpc = n_chunks // _SC_CORES
    mesh = plsc.VectorSubcoreMesh(core_axis_name="c", subcore_axis_name="s")
    out_type = tuple(
        jax.ShapeDtypeStruct((n_chunks * _SC_TILES * cap,), I32)
        for _ in range(3)
    ) + (jax.ShapeDtypeStruct((n_chunks * _SC_TILES * _LANES,), I32),)
    pad = cap + 2 * _LANES
    scratch = [
        pltpu.VMEM((_SCAN_B,), I32),
        pltpu.VMEM((_SCAN_B,), I32),
        pltpu.VMEM((pad,), I32),
        pltpu.VMEM((pad,), I32),
        pltpu.VMEM((pad,), I32),
        pltpu.VMEM((_LANES,), I32),
    ]
    trash = cap + _LANES

    def body(src_h, dst_h, osrc, odst, oeid, ocnt,
             sblk, dblk, ssel, dsel, esel, cnt_v):
        c = lax.axis_index("c")
        t = lax.axis_index("s")
        e0 = t * ept
        lane = lax.iota(I32, _LANES)
        for kk in range(kpc):
            chunk = c * kpc + kk
            lo = chunk * nc
            region = (chunk * _SC_TILES + t) * cap

            def vstep(j, cur, base=None):
                dv = dblk[pl.ds(j * _LANES, _LANES)]
                sv = sblk[pl.ds(j * _LANES, _LANES)]
                dl = dv - lo
                mask = (dl >= 0) & (dl < nc)
                ev = base + j * _LANES + lane
                pc = plsc.cumsum(mask.astype(I32))
                pos = jnp.where(mask, cur + pc - 1, trash)
                plsc.store_scatter(dsel, [pos], dl)
                plsc.store_scatter(ssel, [pos], sv)
                plsc.store_scatter(esel, [pos], ev)
                return cur + jnp.max(pc, axis=0)

            def scan_blk(bi, cur):
                base = e0 + bi * _SCAN_B
                pltpu.sync_copy(src_h.at[pl.ds(base, _SCAN_B)], sblk)
                pltpu.sync_copy(dst_h.at[pl.ds(base, _SCAN_B)], dblk)
                return lax.fori_loop(
                    0, _SCAN_B // _LANES,
                    functools.partial(vstep, base=base), cur)

            cur = lax.fori_loop(0, ept // _SCAN_B, scan_blk, 0)
            padn = ((cur + _BLK - 1) // _BLK) * _BLK
            dummy = jnp.full((_LANES,), nc, I32)
            zero16 = jnp.zeros((_LANES,), I32)

            def padstep(cur):
                dsel[pl.ds(cur, _LANES)] = dummy
                ssel[pl.ds(cur, _LANES)] = zero16
                esel[pl.ds(cur, _LANES)] = zero16
                return cur + _LANES

            lax.while_loop(lambda u: u < padn, padstep, cur)
            roff = pl.multiple_of(region, 8)
            pltpu.sync_copy(ssel.at[pl.ds(0, cap)], osrc.at[pl.ds(roff, cap)])
            pltpu.sync_copy(dsel.at[pl.ds(0, cap)], odst.at[pl.ds(roff, cap)])
            pltpu.sync_copy(esel.at[pl.ds(0, cap)], oeid.at[pl.ds(roff, cap)])
            cnt_v[...] = jnp.full((_LANES,), padn // _BLK, I32)
            coff = pl.multiple_of((chunk * _SC_TILES + t) * _LANES, 8)
            pltpu.sync_copy(cnt_v, ocnt.at[pl.ds(coff, _LANES)])

    return pl.kernel(body, out_type=out_type, mesh=mesh,
                     scratch_types=scratch, compiler_params=_SC_PARAMS)


def _make_agg(n_nodes, n_edges, n_chunks, hid, spmem_table=False):
    ept = n_edges // _SC_TILES
    cap = ept
    nc = n_nodes // n_chunks
    kpc = n_chunks // _SC_CORES
    nq = hid // _LANES
    nzb = nc // _ZROWS
    ntb = n_nodes // _ZROWS
    mesh = plsc.VectorSubcoreMesh(core_axis_name="c", subcore_axis_name="s")
    out_type = jax.ShapeDtypeStruct((n_nodes, hid), F32)
    scratch = [
        pltpu.VMEM_SHARED((nc + _LANES, hid), F32),
        pltpu.VMEM((_BLK, hid), F32),
        pltpu.VMEM((_BLK, hid), F32),
        [pltpu.VMEM((_BLK,), I32) for _ in range(3)],
        [pltpu.VMEM((_BLK,), I32) for _ in range(3)],
        pltpu.VMEM((_LANES,), I32),
        pltpu.SemaphoreType.DMA,
        pltpu.SemaphoreType.DMA,
        pltpu.SemaphoreType.DMA,
        pltpu.SemaphoreType.DMA,
    ]
    if spmem_table:
        scratch.append(pltpu.VMEM_SHARED((n_nodes, hid), F32))

    def body(pre_s, pre_e, lsrc, ldst, leid, lcnt, out,
             agg, buf, buf2, set0, set1, cnt_v, lsem0, lsem1, esem, ssem,
             *maybe_tab):
        stab = maybe_tab[0] if spmem_table else None
        c = lax.axis_index("c")
        t = lax.axis_index("s")
        if spmem_table:
            for b in range(-(-ntb // _SC_TILES)):
                bid = t + b * _SC_TILES
                @pl.when(bid < ntb)
                def _():
                    r0 = pl.multiple_of(bid * _ZROWS, 8)
                    pltpu.sync_copy(pre_s.at[pl.ds(r0, _ZROWS)],
                                    stab.at[pl.ds(bid * _ZROWS, _ZROWS)])
            plsc.subcore_barrier()
        zv = jnp.zeros((_LANES,), F32)
        sets = (set0, set1)
        lsems = (lsem0, lsem1)
        hbm_lists = (lsrc, ldst, leid)

        def issue_lists(ph, base):
            for hl, vb in zip(hbm_lists, sets[ph]):
                pltpu.async_copy(hl.at[pl.ds(base, _BLK)], vb, lsems[ph])

        def wait_lists(ph):
            for hl, vb in zip(hbm_lists, sets[ph]):
                pltpu.make_async_copy(hl.at[pl.ds(0, _BLK)], vb,
                                      lsems[ph]).wait()

        for kk in range(kpc):
            chunk = c * kpc + kk
            region = (chunk * _SC_TILES + t) * cap
            for r in range(_ZROWS):
                for q in range(nq):
                    buf[r, pl.ds(q * _LANES, _LANES)] = zv
            for b in range(-(-nzb // _SC_TILES)):
                bid = t + b * _SC_TILES
                @pl.when(bid < nzb)
                def _():
                    pltpu.sync_copy(buf.at[pl.ds(0, _ZROWS)],
                                    agg.at[pl.ds(bid * _ZROWS, _ZROWS)])
            plsc.subcore_barrier()

            coff = pl.multiple_of((chunk * _SC_TILES + t) * _LANES, 8)
            pltpu.sync_copy(lcnt.at[pl.ds(coff, _LANES)], cnt_v)
            nblk = jnp.max(cnt_v[...], axis=0)

            @pl.when(nblk > 0)
            def _():
                issue_lists(0, pl.multiple_of(region, 8))

            def pair(pr, carry):
                for ph in range(2):
                    b = 2 * pr + ph
                    sidx, didx, eidx = sets[ph]

                    @pl.when(b < nblk)
                    def _():
                        wait_lists(ph)

                        @pl.when(b + 1 < nblk)
                        def _():
                            issue_lists(
                                1 - ph,
                                pl.multiple_of(region + (b + 1) * _BLK, 8))

                        de = pltpu.async_copy(pre_e.at[eidx], buf, esem)
                        if spmem_table:
                            dg = pltpu.async_copy(stab.at[sidx], buf2, ssem)
                        else:
                            dg = pltpu.async_copy(pre_s.at[sidx], buf2, ssem)
                        de.wait()
                        dg.wait()

                        @plsc.parallel_loop(0, _BLK, 1, unroll=8)
                        def _(i):
                            for q in range(nq):
                                sl = pl.ds(q * _LANES, _LANES)
                                buf[i, sl] = jnp.maximum(
                                    buf[i, sl] + buf2[i, sl], 0.0)

                        pltpu.sync_copy(buf, agg.at[didx], add=True)
                return carry

            lax.fori_loop(0, (nblk + 1) // 2, pair, 0)
            plsc.subcore_barrier()

            for b in range(-(-nzb // _SC_TILES)):
                bid = t + b * _SC_TILES
                @pl.when(bid < nzb)
                def _():
                    r0 = bid * _ZROWS
                    pltpu.sync_copy(
                        agg.at[pl.ds(r0, _ZROWS)],
                        out.at[pl.ds(pl.multiple_of(chunk * nc + r0, 8),
                                     _ZROWS)])
            plsc.subcore_barrier()

    return pl.kernel(body, out_type=out_type, mesh=mesh,
                     scratch_types=scratch, compiler_params=_SC_PARAMS)


def _conv(p, x, graph, pre_e, add):
    wm, _ = p['msg']
    h = wm.shape[1]
    pre_s = _linear(x, wm[:h], jnp.zeros((h,), F32))
    agg = graph['agg_fn'](pre_s, pre_e, *graph['lists'])
    return _conv_tail(x, agg, add, p)


def _pre_edge(ea, p):
    wm, bm = p['msg']
    h = wm.shape[1]
    return _linear(ea, wm[h:], bm)


def kernel(node_feature, edge_index, edge_feature, line_node_feature,
           line_edge_index, line_edge_feature, global_state, group_size,
           cell, coords, params):
    p = params
    n_nodes = node_feature.shape[0]
    batch = global_state.shape[0]

    src, dst = edge_index[0], edge_index[1]
    lsrc, ldst = line_edge_index[0], line_edge_index[1]

    node_res, meta_node = _node_stage(node_feature, p['node_linear'],
                                      p['node_ff'])
    meta_edge = _residual_ff(edge_feature, p['edge_linear'], p['edge_ff'])
    line_meta_node = _residual_ff(line_node_feature, p['line_node_linear'],
                                  p['line_node_ff'])
    line_meta_edge = _residual_ff(line_edge_feature, p['line_edge_linear'],
                                  p['line_edge_ff'])

    def tiny_residual(f, lin_p, ff_p):
        (wl, bl), ff = lin_p, ff_p
        y = f @ wl + bl
        hdn = f
        for (w0, b0) in ff[:-1]:
            hdn = jnp.maximum(hdn @ w0 + b0, 0.0)
        return y + hdn @ ff[-1][0] + ff[-1][1]

    meta_state = tiny_residual(global_state, p['state_linear'], p['state_ff'])
    meta_state = jnp.repeat(meta_state, group_size,
                            total_repeat_length=n_nodes)
    meta_cell = tiny_residual(cell, p['cell_linear'], p['cell_ff'])
    cell_mean = jnp.mean(meta_cell)
    meta_coords = _residual_ff(
        jnp.pad(coords, ((0, 0), (0, 5))),
        (jnp.pad(p['coord_linear'][0], ((0, 5), (0, 0))), p['coord_linear'][1]),
        ((jnp.pad(p['coord_ff'][0][0], ((0, 5), (0, 0))), p['coord_ff'][0][1]),
         p['coord_ff'][1]))

    crystal_add = meta_state[:, None] + meta_coords + cell_mean

    hid = 64
    n_ln = line_node_feature.shape[0]
    line_graph = {
        'lists': _make_compact(n_ln, lsrc.shape[0], 10)(lsrc, ldst),
        'agg_fn': _make_agg(n_ln, lsrc.shape[0], 10, hid),
    }
    node_graph = {
        'lists': _make_compact(n_nodes, src.shape[0], 2)(src, dst),
        'agg_fn': _make_agg(n_nodes, src.shape[0], 2, hid,
                            spmem_table=True),
    }

    n_layers = len(p['gcs'])
    for i in range(n_layers):
        line_pre_e = _pre_edge(line_meta_edge, p['line_gcs'][i])
        line_meta_node = _conv(p['line_gcs'][i], line_meta_node, line_graph,
                               line_pre_e, None)
        pre_e = _pre_edge(meta_edge, p['gcs'][i])
        meta_node = _conv(p['gcs'][i], meta_node, node_graph, pre_e,
                          crystal_add)

    fp = p['final_gcs'][n_layers - 1]
    final_pre_e = _pre_edge(line_meta_node, fp)
    final_node = _conv(fp, meta_node, node_graph, final_pre_e, crystal_add)
    return node_res + final_node
